# baseline (speedup 1.0000x reference)
import functools

import jax
import jax.numpy as jnp
from jax import lax
from jax.experimental import pallas as pl
from jax.experimental.pallas import tpu as pltpu
from jax.experimental.pallas import tpu_sc as plsc

EPS = 1e-06

_PERMS = (
    (2, 1, 0),
    (1, 2, 0),
    (0, 1, 2),
    (1, 0, 2),
    (2, 0, 1),
    (0, 1, 2),
    (0, 2, 1),
    (0, 1, 2),
)

_L = 16
_NC = 2
_NS = 16
_NW = _NC * _NS
_DQ = 33
_ROW = 450
_CHUNK = 128
_TCB = 1024


def _round_to_bf16(x):
    u = lax.bitcast_convert_type(x, jnp.int32)
    lsb = lax.shift_right_logical(u, 16) & jnp.int32(1)
    r = (u + jnp.int32(0x7FFF) + lsb) & jnp.int32(-65536)
    return lax.bitcast_convert_type(r, jnp.float32)


def _tc_pid_body(q_ref, k_ref, ikn_ref, pid_ref):
    q = _round_to_bf16(q_ref[...])
    s = []
    for m in range(3):
        km = k_ref[m, :_DQ][None, :]
        s.append(jnp.sum(q * km, axis=1) * ikn_ref[m, 0])
    zero = jnp.zeros(s[0].shape, jnp.int32)
    pid_ref[...] = (
        jnp.where(s[0] >= s[1], jnp.int32(4), zero)
        + jnp.where(s[0] >= s[2], jnp.int32(2), zero)
        + jnp.where(s[1] >= s[2], jnp.int32(1), zero)
    )


def _sc_body(pid_hbm, table_hbm, out_hbm, tabv, idxv, chunkbuf, qw, nch):
    wid = lax.axis_index("s") * _NC + lax.axis_index("c")
    qbase = wid * qw

    pltpu.sync_copy(pid_hbm.at[pl.ds(qbase, qw)], idxv)
    pltpu.sync_copy(table_hbm, tabv)

    nfull = _ROW // _L
    tail = _ROW - _L
    for ch in range(nch):
        def copyg(g, carry):
            pv = idxv[pl.ds(ch * _CHUNK + g * _L, _L)]
            for lane in range(_L):
                p = pv[lane]
                dst = (g * _L + lane) * _ROW
                for j in range(nfull):
                    chunkbuf[pl.ds(dst + j * _L, _L)] = tabv[p, pl.ds(j * _L, _L)]
                chunkbuf[pl.ds(dst + tail, _L)] = tabv[p, pl.ds(tail, _L)]
            return carry

        lax.fori_loop(0, _CHUNK // _L, copyg, 0)
        pltpu.sync_copy(
            chunkbuf,
            out_hbm.at[pl.ds((qbase + ch * _CHUNK) * _ROW, _CHUNK * _ROW)],
        )


def kernel(queries, mem_keys, mem_values, top_num):
    del top_num
    bsz, dq = queries.shape
    m = mem_keys.shape[0]
    assert (m, dq) == (3, _DQ)
    qw = bsz // _NW
    nch = qw // _CHUNK

    kpad = jnp.zeros((8, 128), jnp.float32)
    kpad = kpad.at[:m, :dq].set(_round_to_bf16(mem_keys))
    knorm = jnp.maximum(jnp.linalg.norm(mem_keys, axis=1), EPS)
    ikn = jnp.zeros((8, 128), jnp.float32)
    ikn = ikn.at[:m, 0].set((1.0 / knorm).astype(jnp.float32))
    table = jnp.stack(
        [mem_values[jnp.array(p)].reshape(-1) for p in _PERMS]
    ).astype(jnp.float32)

    pid = pl.pallas_call(
        _tc_pid_body,
        grid=(bsz // _TCB,),
        in_specs=[
            pl.BlockSpec((_TCB, dq), lambda i: (i, 0)),
            pl.BlockSpec((8, 128), lambda i: (0, 0)),
            pl.BlockSpec((8, 128), lambda i: (0, 0)),
        ],
        out_specs=pl.BlockSpec((_TCB,), lambda i: (i,)),
        out_shape=jax.ShapeDtypeStruct((bsz,), jnp.int32),
    )(queries, kpad, ikn)

    mesh = plsc.VectorSubcoreMesh(core_axis_name="c", subcore_axis_name="s")
    run = pl.kernel(
        functools.partial(_sc_body, qw=qw, nch=nch),
        out_type=jax.ShapeDtypeStruct((bsz * _ROW,), jnp.float32),
        mesh=mesh,
        scratch_types=[
            pltpu.VMEM((8, _ROW), jnp.float32),
            pltpu.VMEM((qw,), jnp.int32),
            pltpu.VMEM((_CHUNK * _ROW,), jnp.float32),
        ],
    )
    out = run(pid, table)
    return out.reshape(bsz, m, mem_values.shape[1], mem_values.shape[2])


# speedup vs baseline: 54.8786x; 54.8786x over previous
import jax
import jax.numpy as jnp
from jax import lax
from jax.experimental import pallas as pl

EPS = 1e-06

_PERMS = (
    (2, 1, 0),
    (1, 2, 0),
    (0, 1, 2),
    (1, 0, 2),
    (2, 0, 1),
    (0, 1, 2),
    (0, 2, 1),
    (0, 1, 2),
)

_DQ = 33
_NV = 50
_BQ = 2048


def _round_to_bf16(x):
    u = lax.bitcast_convert_type(x, jnp.int32)
    lsb = lax.shift_right_logical(u, 16) & jnp.int32(1)
    r = (u + jnp.int32(0x7FFF) + lsb) & jnp.int32(-65536)
    return lax.bitcast_convert_type(r, jnp.float32)


def _fused_body(q_ref, k_ref, ikn_ref, v_ref, out_ref):
    q = _round_to_bf16(q_ref[...])
    s = []
    for mm in range(3):
        km = k_ref[mm, :_DQ][None, :]
        s.append(jnp.sum(q * km, axis=1, keepdims=True) * ikn_ref[mm, 0])
    zero = jnp.zeros((_BQ, 1), jnp.int32)
    pid = (
        jnp.where(s[0] >= s[1], jnp.int32(4), zero)
        + jnp.where(s[0] >= s[2], jnp.int32(2), zero)
        + jnp.where(s[1] >= s[2], jnp.int32(1), zero)
    )
    pid_t = jnp.transpose(pid, (1, 0))

    for r in range(3):
        g = jnp.full((1, _BQ), _PERMS[0][r], jnp.int32)
        for k in range(1, 8):
            g = jnp.where(pid_t == k, jnp.int32(_PERMS[k][r]), g)
        c0 = jnp.broadcast_to(g == 0, (_NV, _BQ))
        c1 = jnp.broadcast_to(g == 1, (_NV, _BQ))
        for j in range(3):
            v0 = jnp.broadcast_to(v_ref[0, j, :_NV, :1], (_NV, _BQ))
            v1 = jnp.broadcast_to(v_ref[1, j, :_NV, :1], (_NV, _BQ))
            v2 = jnp.broadcast_to(v_ref[2, j, :_NV, :1], (_NV, _BQ))
            out_ref[r, j] = jnp.where(c0, v0, jnp.where(c1, v1, v2))


def kernel(queries, mem_keys, mem_values, top_num):
    del top_num
    bsz, dq = queries.shape
    m, nv, _ = mem_values.shape
    assert (m, dq, nv) == (3, _DQ, _NV)

    kpad = jnp.zeros((8, 128), jnp.float32)
    kpad = kpad.at[:m, :dq].set(_round_to_bf16(mem_keys))
    knorm = jnp.maximum(jnp.linalg.norm(mem_keys, axis=1), EPS)
    ikn = jnp.zeros((8, 128), jnp.float32)
    ikn = ikn.at[:m, 0].set((1.0 / knorm).astype(jnp.float32))
    vjm = jnp.zeros((3, 3, 56, 128), jnp.float32)
    vjm = vjm.at[:, :, :nv, :].set(
        jnp.transpose(mem_values, (0, 2, 1))[:, :, :, None]
    )

    out4 = pl.pallas_call(
        _fused_body,
        grid=(bsz // _BQ,),
        in_specs=[
            pl.BlockSpec((_BQ, dq), lambda i: (i, 0)),
            pl.BlockSpec((8, 128), lambda i: (0, 0)),
            pl.BlockSpec((8, 128), lambda i: (0, 0)),
            pl.BlockSpec((3, 3, 56, 128), lambda i: (0, 0, 0, 0)),
        ],
        out_specs=pl.BlockSpec((3, 3, _NV, _BQ), lambda i: (0, 0, 0, i)),
        out_shape=jax.ShapeDtypeStruct((3, 3, _NV, bsz), jnp.float32),
    )(queries, kpad, ikn, vjm)

    return jnp.transpose(out4, (3, 0, 2, 1))
